# Initial kernel scaffold; baseline (speedup 1.0000x reference)
import jax
import jax.numpy as jnp
from jax.experimental import pallas as pl


def kernel(hidden_states, norm_weight, W_router, W_gate, W_up, W_down):
    raise NotImplementedError("write your pallas kernel here")


# speedup vs baseline: 1.5864x; 1.5864x over previous
import functools

import jax
import jax.numpy as jnp
from jax import lax
from jax.experimental import pallas as pl
from jax.experimental.pallas import tpu as pltpu
from jax.experimental.pallas import tpu_sc as plsc

HIDDEN = 2048
E = 8
TOPK = 2
DFF = 1408
EPS = 1e-06
T = 2048

BLK = 256
NT = 24
NROWS = NT * BLK
NROWS_P = NROWS + BLK
EH = E // 2

NW = 32
TOK_W = T // NW
CH = 16
NCH = TOK_W // CH


def _norm_router_body(x_ref, nw_ref, wr_ref, xn_ref, lg_ref):
    x = x_ref[...]
    var = jnp.mean(x * x, axis=1, keepdims=True)
    xn = x * lax.rsqrt(var + EPS) * nw_ref[...]
    xn_ref[...] = xn
    lg_ref[...] = jnp.dot(xn, wr_ref[...], preferred_element_type=jnp.float32)


def _norm_router(x, norm_weight, w_router):
    nblk = 8
    rb = T // nblk
    return pl.pallas_call(
        _norm_router_body,
        grid=(nblk,),
        in_specs=[
            pl.BlockSpec((rb, HIDDEN), lambda i: (i, 0)),
            pl.BlockSpec((1, HIDDEN), lambda i: (0, 0)),
            pl.BlockSpec((HIDDEN, E), lambda i: (0, 0)),
        ],
        out_specs=[
            pl.BlockSpec((rb, HIDDEN), lambda i: (i, 0)),
            pl.BlockSpec((rb, E), lambda i: (i, 0)),
        ],
        out_shape=[
            jax.ShapeDtypeStruct((T, HIDDEN), jnp.float32),
            jax.ShapeDtypeStruct((T, E), jnp.float32),
        ],
    )(x, norm_weight.reshape(1, HIDDEN), w_router)


def _route_body(lg_ref, d0_ref, d1_ref, w0_ref, w1_ref, te_ref, tot_ref):
    lg = lg_ref[...]
    eidx = lax.broadcasted_iota(jnp.int32, (T, E), 1)
    m1 = jnp.max(lg, axis=1, keepdims=True)
    a1 = jnp.min(jnp.where(lg == m1, eidx, E), axis=1, keepdims=True)
    lg2 = jnp.where(eidx == a1, -1e30, lg)
    m2 = jnp.max(lg2, axis=1, keepdims=True)
    a2 = jnp.min(jnp.where(lg2 == m2, eidx, E), axis=1, keepdims=True)
    e2 = jnp.exp(m2 - m1)
    w0 = 1.0 / (1.0 + e2)
    w1 = 1.0 - w0
    oh1 = (eidx == a1)
    oh2 = (eidx == a2)
    oh = (oh1 | oh2).astype(jnp.float32)

    cb = 256
    r = lax.broadcasted_iota(jnp.int32, (cb, cb), 0)
    c = lax.broadcasted_iota(jnp.int32, (cb, cb), 1)
    tri = (c < r).astype(jnp.float32)
    colsum = jnp.zeros((1, E), jnp.float32)
    chunks = []
    for k in range(T // cb):
        ohc = lax.slice(oh, (k * cb, 0), ((k + 1) * cb, E))
        chunks.append(jnp.dot(tri, ohc, preferred_element_type=jnp.float32)
                      + colsum)
        colsum = colsum + jnp.sum(ohc, axis=0, keepdims=True)
    pos = jnp.concatenate(chunks, axis=0)

    sizes = colsum
    ntiles = jnp.floor((sizes + (BLK - 1)) * (1.0 / BLK))
    tri8 = (lax.broadcasted_iota(jnp.int32, (E, E), 0)
            < lax.broadcasted_iota(jnp.int32, (E, E), 1)).astype(jnp.float32)
    cumt = jnp.dot(ntiles, tri8, preferred_element_type=jnp.float32)
    total = jnp.sum(ntiles).astype(jnp.int32)
    rowoff = cumt * BLK

    pos1 = jnp.sum(jnp.where(oh1, pos, 0.0), axis=1)
    off1 = jnp.sum(jnp.where(oh1, rowoff, 0.0), axis=1)
    pos2 = jnp.sum(jnp.where(oh2, pos, 0.0), axis=1)
    off2 = jnp.sum(jnp.where(oh2, rowoff, 0.0), axis=1)
    d0_ref[...] = (off1 + pos1).astype(jnp.int32).reshape(1, T)
    d1_ref[...] = (off2 + pos2).astype(jnp.int32).reshape(1, T)
    w0_ref[...] = w0.reshape(1, T)
    w1_ref[...] = w1.reshape(1, T)

    ti = lax.broadcasted_iota(jnp.int32, (NT, E), 0)
    ti = jnp.minimum(ti, total - 1)
    cumtb = jnp.broadcast_to(cumt.astype(jnp.int32), (NT, E))
    te = jnp.sum((cumtb <= ti).astype(jnp.int32), axis=1) - 1
    te_ref[...] = te.reshape(1, NT)
    tiles_a = lax.slice(cumt, (0, EH), (1, EH + 1)).astype(jnp.int32)
    tot_ref[...] = jnp.concatenate(
        [tiles_a, jnp.broadcast_to(total, (1, 1))], axis=1)


def _route(logits):
    outs = pl.pallas_call(
        _route_body,
        out_shape=[
            jax.ShapeDtypeStruct((1, T), jnp.int32),
            jax.ShapeDtypeStruct((1, T), jnp.int32),
            jax.ShapeDtypeStruct((1, T), jnp.float32),
            jax.ShapeDtypeStruct((1, T), jnp.float32),
            jax.ShapeDtypeStruct((1, NT), jnp.int32),
            jax.ShapeDtypeStruct((1, 2), jnp.int32),
        ],
    )(logits)
    d0, d1, w0, w1, te, tot = outs
    return (d0.reshape(T), d1.reshape(T), w0.reshape(T), w1.reshape(T),
            te.reshape(NT), tot.reshape(2))


def _dispatch_body(xn_hbm, d0_hbm, d1_hbm, buf_hbm,
                   idx0_v, idx1_v, rows_v, rsem, w0sem, w1sem):
    nc = 2
    wid = lax.axis_index("s") * nc + lax.axis_index("c")
    base = wid * TOK_W

    def read(ci, b):
        tb = base + ci * CH
        pltpu.sync_copy(d0_hbm.at[pl.ds(tb, CH)], idx0_v.at[b])
        pltpu.sync_copy(d1_hbm.at[pl.ds(tb, CH)], idx1_v.at[b])
        pltpu.make_async_copy(xn_hbm.at[pl.ds(tb, CH)], rows_v.at[b],
                              rsem.at[b]).start()

    read(0, 0)
    for ci in range(NCH):
        b = ci & 1
        pltpu.make_async_copy(xn_hbm.at[pl.ds(0, CH)], rows_v.at[b],
                              rsem.at[b]).wait()
        cp0 = pltpu.make_async_copy(rows_v.at[b], buf_hbm.at[idx0_v.at[b]],
                                    w0sem.at[b])
        cp1 = pltpu.make_async_copy(rows_v.at[b], buf_hbm.at[idx1_v.at[b]],
                                    w1sem.at[b])
        cp0.start()
        cp1.start()
        if ci + 1 < NCH:
            nb = b ^ 1
            if ci >= 1:
                pltpu.make_async_copy(rows_v.at[nb], buf_hbm.at[idx0_v.at[nb]],
                                      w0sem.at[nb]).wait()
                pltpu.make_async_copy(rows_v.at[nb], buf_hbm.at[idx1_v.at[nb]],
                                      w1sem.at[nb]).wait()
            read(ci + 1, nb)
    for b in (0, 1):
        pltpu.make_async_copy(rows_v.at[b], buf_hbm.at[idx0_v.at[b]],
                              w0sem.at[b]).wait()
        pltpu.make_async_copy(rows_v.at[b], buf_hbm.at[idx1_v.at[b]],
                              w1sem.at[b]).wait()


def _dispatch(xn, d0, d1):
    mesh = plsc.VectorSubcoreMesh(core_axis_name="c", subcore_axis_name="s")
    k = pl.kernel(
        _dispatch_body,
        out_type=jax.ShapeDtypeStruct((NROWS, HIDDEN), jnp.float32),
        mesh=mesh,
        scratch_types=[
            pltpu.VMEM((2, CH), jnp.int32),
            pltpu.VMEM((2, CH), jnp.int32),
            pltpu.VMEM((2, CH, HIDDEN), jnp.float32),
            pltpu.SemaphoreType.DMA((2,)),
            pltpu.SemaphoreType.DMA((2,)),
            pltpu.SemaphoreType.DMA((2,)),
        ],
    )
    return k(xn, d0, d1)


def _ffn_body(te_ref, tot_ref, buf_ref, wg_ref, wu_ref, wd_ref, y_ref):
    i = pl.program_id(0)

    @pl.when(i < tot_ref[1])
    def _():
        x = buf_ref[...].astype(jnp.bfloat16)
        g = jnp.dot(x, wg_ref[0], preferred_element_type=jnp.float32)
        u = jnp.dot(x, wu_ref[0], preferred_element_type=jnp.float32)
        h = (g * lax.logistic(g) * u).astype(jnp.bfloat16)
        y_ref[...] = jnp.dot(h, wd_ref[0], preferred_element_type=jnp.float32)


def _ffn(te, tots, buf, wg, wu, wd):
    grid_spec = pltpu.PrefetchScalarGridSpec(
        num_scalar_prefetch=2,
        grid=(NT,),
        in_specs=[
            pl.BlockSpec((BLK, HIDDEN), lambda i, te, tt: (i, 0)),
            pl.BlockSpec((1, HIDDEN, DFF), lambda i, te, tt: (te[i], 0, 0)),
            pl.BlockSpec((1, HIDDEN, DFF), lambda i, te, tt: (te[i], 0, 0)),
            pl.BlockSpec((1, DFF, HIDDEN), lambda i, te, tt: (te[i], 0, 0)),
        ],
        out_specs=pl.BlockSpec((BLK, HIDDEN), lambda i, te, tt: (i, 0)),
    )
    return pl.pallas_call(
        _ffn_body,
        grid_spec=grid_spec,
        out_shape=jax.ShapeDtypeStruct((NROWS, HIDDEN), jnp.float32),
        compiler_params=pltpu.CompilerParams(
            dimension_semantics=("arbitrary",),
            allow_input_fusion=(False, False, False, True, True, True),
        ),
    )(te, tots, buf, wg, wu, wd)


CCH = 8
CNCH = TOK_W // CCH


def _combine_body(y_hbm, res_hbm, d0_hbm, d1_hbm, w0_hbm, w1_hbm, out_hbm,
                  idx0_v, idx1_v, w0_v, w1_v, res_v, y0_v, y1_v,
                  g0sem, g1sem, rsem, wsem):
    nc = 2
    wid = lax.axis_index("s") * nc + lax.axis_index("c")
    base = wid * TOK_W
    nd = HIDDEN // 16

    pltpu.sync_copy(d0_hbm.at[pl.ds(base, TOK_W)], idx0_v)
    pltpu.sync_copy(d1_hbm.at[pl.ds(base, TOK_W)], idx1_v)
    pltpu.sync_copy(w0_hbm.at[pl.ds(base, TOK_W)], w0_v)
    pltpu.sync_copy(w1_hbm.at[pl.ds(base, TOK_W)], w1_v)

    def read(ci, b):
        tb = base + ci * CCH
        pltpu.make_async_copy(y_hbm.at[idx0_v.at[pl.ds(ci * CCH, CCH)]],
                              y0_v.at[b], g0sem.at[b]).start()
        pltpu.make_async_copy(y_hbm.at[idx1_v.at[pl.ds(ci * CCH, CCH)]],
                              y1_v.at[b], g1sem.at[b]).start()
        pltpu.make_async_copy(res_hbm.at[pl.ds(tb, CCH)], res_v.at[b],
                              rsem.at[b]).start()

    read(0, 0)
    for ci in range(CNCH):
        b = ci & 1
        tb = base + ci * CCH
        pltpu.make_async_copy(y_hbm.at[pl.ds(0, CCH)], y0_v.at[b],
                              g0sem.at[b]).wait()
        pltpu.make_async_copy(y_hbm.at[pl.ds(0, CCH)], y1_v.at[b],
                              g1sem.at[b]).wait()
        pltpu.make_async_copy(res_hbm.at[pl.ds(0, CCH)], res_v.at[b],
                              rsem.at[b]).wait()
        if ci + 1 < CNCH:
            nb = b ^ 1
            if ci >= 1:
                pltpu.make_async_copy(res_v.at[nb],
                                      out_hbm.at[pl.ds(0, CCH)],
                                      wsem.at[nb]).wait()
            read(ci + 1, nb)
        w0a = w0_v[pl.ds((ci // 2) * 16, 16)]
        w1a = w1_v[pl.ds((ci // 2) * 16, 16)]
        lo = (ci % 2) * CCH
        for t in range(CCH):
            w0b = jnp.broadcast_to(w0a[lo + t], (16,))
            w1b = jnp.broadcast_to(w1a[lo + t], (16,))

            def body(d, _, t=t, w0b=w0b, w1b=w1b):
                s = pl.ds(d * 16, 16)
                res_v[b, t, s] = (res_v[b, t, s] + w0b * y0_v[b, t, s]
                                  + w1b * y1_v[b, t, s])
                return 0

            lax.fori_loop(0, nd, body, 0, unroll=8)
        pltpu.make_async_copy(res_v.at[b], out_hbm.at[pl.ds(tb, CCH)],
                              wsem.at[b]).start()
    for b in (0, 1):
        pltpu.make_async_copy(res_v.at[b], out_hbm.at[pl.ds(0, CCH)],
                              wsem.at[b]).wait()


def _combine(y, residual, d0, d1, w0, w1):
    mesh = plsc.VectorSubcoreMesh(core_axis_name="c", subcore_axis_name="s")
    k = pl.kernel(
        _combine_body,
        out_type=jax.ShapeDtypeStruct((T, HIDDEN), jnp.float32),
        mesh=mesh,
        scratch_types=[
            pltpu.VMEM((TOK_W,), jnp.int32),
            pltpu.VMEM((TOK_W,), jnp.int32),
            pltpu.VMEM((TOK_W,), jnp.float32),
            pltpu.VMEM((TOK_W,), jnp.float32),
            pltpu.VMEM((2, CCH, HIDDEN), jnp.float32),
            pltpu.VMEM((2, CCH, HIDDEN), jnp.float32),
            pltpu.VMEM((2, CCH, HIDDEN), jnp.float32),
            pltpu.SemaphoreType.DMA((2,)),
            pltpu.SemaphoreType.DMA((2,)),
            pltpu.SemaphoreType.DMA((2,)),
            pltpu.SemaphoreType.DMA((2,)),
        ],
    )
    return k(y, residual, d0, d1, w0, w1)


def kernel(hidden_states, norm_weight, W_router, W_gate, W_up, W_down):
    b, s, d = hidden_states.shape
    wgb = W_gate.astype(jnp.bfloat16)
    wub = W_up.astype(jnp.bfloat16)
    wdb = W_down.astype(jnp.bfloat16)
    x = hidden_states.reshape(T, HIDDEN)
    xn, logits = _norm_router(x, norm_weight, W_router)
    d0, d1, w0, w1, te, tots = _route(logits)
    buf = _dispatch(xn, d0, d1)
    y = _ffn(te, tots, buf, wgb, wub, wdb)
    out = _combine(y, x, d0, d1, w0, w1)
    return out.reshape(b, s, d)
